# baseline (device time: 25818 ns/iter reference)
import jax
import jax.numpy as jnp
from jax import lax
from jax.experimental import pallas as pl
from jax.experimental.pallas import tpu as pltpu

N_DEV = 4
B, SQ, D = 2, 256, 768
HQ_LOC, DH = 8, 64
SKV = 512
DQ_LOC = HQ_LOC * DH
HD = D // 2
QD = D // 4


def kernel(x, Wq, Wo, K_ext, V_ext):
    def body(x_ref, wq_ref, wo_ref, k_ref, v_ref, out_ref,
             sendH, recvH, sendQ, recvQ, attn_ref,
             ssemH, rsemH, ssemQ, rsemQ):
        my = lax.axis_index("i")
        peers = [jnp.bitwise_xor(my, 1), 3 - my]

        barrier_sem = pltpu.get_barrier_semaphore()
        for stage in range(2):
            pl.semaphore_signal(barrier_sem, inc=1, device_id=(peers[stage],),
                                device_id_type=pl.DeviceIdType.MESH)
        pl.semaphore_wait(barrier_sem, 2)

        wq = wq_ref[...].astype(jnp.bfloat16)
        wo = wo_ref[...].astype(jnp.bfloat16)

        def prep(b):
            xb = x_ref[b].astype(jnp.bfloat16)
            qb = jnp.dot(xb, wq, preferred_element_type=jnp.float32)
            qb = (qb * 0.125).astype(jnp.bfloat16)
            kb = k_ref[b].astype(jnp.bfloat16)
            vb = v_ref[b].astype(jnp.bfloat16)
            return qb, kb, vb

        def heads(qkv, h0, h1):
            qb, kb, vb = qkv
            for h in range(h0, h1):
                q = qb[:, h * DH:(h + 1) * DH]
                k = kb[:, h * DH:(h + 1) * DH]
                v = vb[:, h * DH:(h + 1) * DH]
                s = lax.dot_general(q, k, (((1,), (1,)), ((), ())),
                                    preferred_element_type=jnp.float32)
                p = jnp.exp(s)
                denom = jnp.sum(p, axis=-1, keepdims=True)
                o = jnp.dot(p.astype(jnp.bfloat16), v,
                            preferred_element_type=jnp.float32)
                o = o * (1.0 / denom)
                attn_ref[:, h * DH:(h + 1) * DH] = o.astype(jnp.bfloat16)

        def xchg(sref, rref, ssem, rsem, slot, link):
            rdma = pltpu.make_async_remote_copy(
                src_ref=sref.at[slot],
                dst_ref=rref.at[slot],
                send_sem=ssem.at[slot],
                recv_sem=rsem.at[slot],
                device_id=(peers[link],),
                device_id_type=pl.DeviceIdType.MESH,
            )
            rdma.start()
            return rdma

        def wo_chunk(lo, width):
            return jnp.dot(attn_ref[...], wo[:, lo:lo + width],
                           preferred_element_type=jnp.float32
                           ).astype(jnp.bfloat16)

        heads(prep(0), 0, HQ_LOC)
        r0 = []
        for i in range(2):
            sendH[i] = wo_chunk(i * HD, HD)
            r0.append(xchg(sendH, recvH, ssemH, rsemH, i, i))

        qkv1 = prep(1)
        heads(qkv1, 0, 4)
        r1 = []
        for i in range(2):
            r0[i].wait()
            sendH[2 + i] = sendH[i] + recvH[i]
            r1.append(xchg(sendH, recvH, ssemH, rsemH, 2 + i, 1 - i))
        heads(qkv1, 4, HQ_LOC)

        q0 = []
        for qi in range(4):
            sendQ[qi] = wo_chunk(qi * QD, QD)
            q0.append(xchg(sendQ, recvQ, ssemQ, rsemQ, qi, qi % 2))

        for i in range(2):
            r1[i].wait()
            out_ref[0, :, i * HD:(i + 1) * HD] = (
                sendH[2 + i].astype(jnp.float32)
                + recvH[2 + i].astype(jnp.float32))

        q1 = []
        for qi in range(4):
            q0[qi].wait()
            sendQ[4 + qi] = sendQ[qi] + recvQ[qi]
            q1.append(xchg(sendQ, recvQ, ssemQ, rsemQ, 4 + qi, 1 - qi % 2))

        for qi in range(4):
            q1[qi].wait()
            out_ref[1, :, qi * QD:(qi + 1) * QD] = (
                sendQ[4 + qi].astype(jnp.float32)
                + recvQ[4 + qi].astype(jnp.float32))

    K_t = K_ext.reshape(B, SKV, DQ_LOC)
    V_t = V_ext.reshape(B, SKV, DQ_LOC)
    return pl.pallas_call(
        body,
        out_shape=jax.ShapeDtypeStruct((B, SQ, D), jnp.float32),
        in_specs=[pl.BlockSpec(memory_space=pltpu.VMEM)] * 5,
        out_specs=pl.BlockSpec(memory_space=pltpu.VMEM),
        scratch_shapes=[
            pltpu.VMEM((4, SQ, HD), jnp.bfloat16),
            pltpu.VMEM((4, SQ, HD), jnp.bfloat16),
            pltpu.VMEM((8, SQ, QD), jnp.bfloat16),
            pltpu.VMEM((8, SQ, QD), jnp.bfloat16),
            pltpu.VMEM((SQ, DQ_LOC), jnp.bfloat16),
            pltpu.SemaphoreType.DMA((4,)),
            pltpu.SemaphoreType.DMA((4,)),
            pltpu.SemaphoreType.DMA((8,)),
            pltpu.SemaphoreType.DMA((8,)),
        ],
        compiler_params=pltpu.CompilerParams(collective_id=0),
    )(x, Wq, Wo, K_t, V_t)
